# baseline (speedup 1.0000x reference)
import jax
import jax.numpy as jnp
from jax.experimental import pallas as pl
from jax.experimental.pallas import tpu as pltpu


def kernel(x, row_W, col_W, pose_W, p):
    b, _, h, w = x.shape
    d = row_W.shape[1]

    def body(p_ref, row_ref, col_ref, pose_ref, pemb_ref, memb_ref):
        bi = pl.program_id(0)
        ci = pl.program_id(1)

        @pl.when((bi == 0) & (ci == 0))
        def _():
            half = jnp.broadcast_to(pose_ref[p_ref[0], :][None, :], (b, d))
            pemb_ref[...] = jnp.concatenate([half, half], axis=1)

        @pl.when(ci == 0)
        def _():
            col_t = col_ref[1:w + 1, :].T
            memb_ref[...] = jnp.broadcast_to(
                col_t[None, :, None, :], (1, d, h, w))

        @pl.when(ci == 1)
        def _():
            row_t = row_ref[1:h + 1, :].T
            memb_ref[...] = jnp.broadcast_to(
                row_t[None, :, :, None], (1, d, h, w))

    grid_spec = pltpu.PrefetchScalarGridSpec(
        num_scalar_prefetch=1,
        grid=(b, 2),
        in_specs=[
            pl.BlockSpec(row_W.shape, lambda bi, ci, p_: (0, 0)),
            pl.BlockSpec(col_W.shape, lambda bi, ci, p_: (0, 0)),
            pl.BlockSpec(pose_W.shape, lambda bi, ci, p_: (0, 0)),
        ],
        out_specs=[
            pl.BlockSpec((b, 2 * d), lambda bi, ci, p_: (0, 0)),
            pl.BlockSpec((1, d, h, w), lambda bi, ci, p_: (bi, ci, 0, 0)),
        ],
    )
    p_emb, m_emb = pl.pallas_call(
        body,
        grid_spec=grid_spec,
        out_shape=[
            jax.ShapeDtypeStruct((b, 2 * d), jnp.float32),
            jax.ShapeDtypeStruct((b, 2 * d, h, w), jnp.float32),
        ],
    )(jnp.reshape(p, (1,)).astype(jnp.int32), row_W, col_W, pose_W)
    return (p_emb, m_emb)
